# Initial kernel scaffold; baseline (speedup 1.0000x reference)
import jax
import jax.numpy as jnp
from jax.experimental import pallas as pl


def kernel(num_features, cat_features, feature_bias, cat_tables, W_num, b_num, W1, b1, W2, b2, W3, b3):
    raise NotImplementedError("write your pallas kernel here")


# speedup vs baseline: 2.2738x; 2.2738x over previous
import functools

import jax
import jax.numpy as jnp
from jax import lax
from jax.experimental import pallas as pl
from jax.experimental.pallas import tpu as pltpu
from jax.experimental.pallas import tpu_sc as plsc

F = 26
V = 100000
E = 32
B = 4096
NUMD = 13
OUT = 128
H1, H2 = 512, 256
D_IN = E + F * E

NC, NS = 2, 16
NW = NC * NS
BPW = B // NW
IPW = BPW * F
LANES = 16

BT = 512


def _sc_gather_body(table, bias_tab, idx_hbm, emb_out, bias_out,
                    idx_v, rows_v, bias_v, sem):
    c = lax.axis_index("c")
    s = lax.axis_index("s")
    wid = s * NC + c
    base = wid * IPW
    pltpu.sync_copy(idx_hbm.at[pl.ds(base, IPW)], idx_v)
    pltpu.async_copy(bias_tab.at[idx_v], bias_v, sem).wait()
    pltpu.sync_copy(bias_v, bias_out.at[pl.ds(base, IPW)])

    def add_off(j, carry):
        p = jnp.arange(LANES, dtype=jnp.int32) + j * LANES
        off = lax.rem(p, F) * V
        idx_v[pl.ds(j * LANES, LANES)] = idx_v[pl.ds(j * LANES, LANES)] + off
        return carry

    lax.fori_loop(0, IPW // LANES, add_off, 0)

    pltpu.async_copy(table.at[idx_v], rows_v, sem).wait()
    pltpu.sync_copy(rows_v, emb_out.at[pl.ds(base, IPW)])


def _sc_gather(table_flat, bias_flat, idx_flat):
    mesh = plsc.VectorSubcoreMesh(core_axis_name="c", subcore_axis_name="s",
                                  num_cores=NC, num_subcores=NS)
    call = pl.kernel(
        _sc_gather_body,
        out_type=(
            jax.ShapeDtypeStruct((B * F, E), jnp.float32),
            jax.ShapeDtypeStruct((B * F,), jnp.float32),
        ),
        mesh=mesh,
        scratch_types=[
            pltpu.VMEM((IPW,), jnp.int32),
            pltpu.VMEM((IPW, E), jnp.float32),
            pltpu.VMEM((IPW,), jnp.float32),
            pltpu.SemaphoreType.DMA,
        ],
        compiler_params=pltpu.CompilerParams(use_tc_tiling_on_sc=False),
    )
    return call(table_flat, bias_flat, idx_flat)


def _mlp_body(xnum, cat, biasr, Wn, bn, W1, b1, W2, b2, W3, b3, out):
    nemb = jnp.dot(xnum[...], Wn[...], preferred_element_type=jnp.float32)
    nemb = nemb + bn[...]
    h = jnp.dot(nemb, W1[0:E, :], preferred_element_type=jnp.float32)
    h = h + jnp.dot(cat[...], W1[E:D_IN, :], preferred_element_type=jnp.float32)
    h = jnp.maximum(h + b1[...], 0.0)
    h = jnp.maximum(jnp.dot(h, W2[...], preferred_element_type=jnp.float32) + b2[...], 0.0)
    deep = jnp.dot(h, W3[...], preferred_element_type=jnp.float32) + b3[...]
    out[:, 0:OUT] = deep
    out[:, OUT:OUT + 1] = jnp.sum(biasr[...], axis=1, keepdims=True)


def _mlp(xnum, cat, biasr, Wn, bn, W1, b1, W2, b2, W3, b3):
    grid = (B // BT,)
    full = lambda shape: pl.BlockSpec(shape, lambda i: (0, 0))
    return pl.pallas_call(
        _mlp_body,
        grid=grid,
        in_specs=[
            pl.BlockSpec((BT, NUMD), lambda i: (i, 0)),
            pl.BlockSpec((BT, F * E), lambda i: (i, 0)),
            pl.BlockSpec((BT, F), lambda i: (i, 0)),
            full((NUMD, E)),
            full((1, E)),
            full((D_IN, H1)),
            full((1, H1)),
            full((H1, H2)),
            full((1, H2)),
            full((H2, OUT)),
            full((1, OUT)),
        ],
        out_specs=pl.BlockSpec((BT, OUT + 1), lambda i: (i, 0)),
        out_shape=jax.ShapeDtypeStruct((B, OUT + 1), jnp.float32),
    )(xnum, cat, biasr, Wn, bn, W1, b1, W2, b2, W3, b3)


def kernel(num_features, cat_features, feature_bias, cat_tables,
           W_num, b_num, W1, b1, W2, b2, W3, b3):
    idx_flat = cat_features.astype(jnp.int32).T.reshape(B * F)
    table_flat = cat_tables.reshape(F * V, E)
    bias_flat = feature_bias.reshape(F * V)

    emb_flat, bias_rows = _sc_gather(table_flat, bias_flat, idx_flat)
    cat_embeds = emb_flat.reshape(B, F * E)
    bias_rows = bias_rows.reshape(B, F)

    return _mlp(num_features, cat_embeds, bias_rows,
                W_num, b_num.reshape(1, E), W1, b1.reshape(1, H1),
                W2, b2.reshape(1, H2), W3, b3.reshape(1, OUT))


# speedup vs baseline: 3.5644x; 1.5676x over previous
import jax
import jax.numpy as jnp
from jax import lax
from jax.experimental import pallas as pl
from jax.experimental.pallas import tpu as pltpu
from jax.experimental.pallas import tpu_sc as plsc

F = 26
V = 100000
E = 32
B = 4096
NUMD = 13
OUT = 128
H1, H2 = 512, 256
D_IN = E + F * E
FE = F * E

NC, NS = 2, 16
NW = NC * NS
RPW = FE // NW
BPW = B // NW

BT = 512


def _sc_gather_body(tabT, bias_tab, idx_hbm, embT_out, first_out,
                    idx_v, idxf_v, gidx_a, gidx_b, dst_a, dst_b,
                    acc_v, bv_v, sem_a, sem_b, sem_c):
    c = lax.axis_index("c")
    s = lax.axis_index("s")
    wid = s * NC + c
    r0 = wid * RPW
    f0 = r0 // E

    pltpu.sync_copy(idx_hbm.at[f0], idx_v.at[0])
    f1 = jnp.minimum(f0 + 1, F - 1)
    pltpu.sync_copy(idx_hbm.at[f1], idx_v.at[1])

    def mk_gidx(t, gidx):
        r = r0 + t
        fsel = r // E - f0
        roff = r * V

        def grp(j, carry):
            gidx[pl.ds(j * 16, 16)] = idx_v[fsel, pl.ds(j * 16, 16)] + roff
            return carry

        lax.fori_loop(0, B // 16, grp, 0)

    def fire(t, gidx, dst, sem):
        mk_gidx(t, gidx)
        pltpu.async_copy(tabT.at[gidx], dst, sem)

    def drain_write(t, gidx, dst, sem):
        pltpu.make_async_copy(tabT.at[gidx], dst, sem).wait()
        pltpu.sync_copy(dst, embT_out.at[r0 + t])

    fire(0, gidx_a, dst_a, sem_a)
    fire(1, gidx_b, dst_b, sem_b)

    def step(t2, carry):
        t = t2 * 2
        drain_write(t, gidx_a, dst_a, sem_a)
        fire(t + 2, gidx_a, dst_a, sem_a)
        drain_write(t + 1, gidx_b, dst_b, sem_b)
        fire(t + 3, gidx_b, dst_b, sem_b)
        return carry

    lax.fori_loop(0, RPW // 2 - 1, step, 0)
    drain_write(RPW - 2, gidx_a, dst_a, sem_a)
    drain_write(RPW - 1, gidx_b, dst_b, sem_b)

    pltpu.sync_copy(idx_hbm.at[:, pl.ds(wid * BPW, BPW)], idxf_v)
    zeros = jnp.zeros((16,), jnp.float32)
    for j in range(BPW // 16):
        acc_v[pl.ds(j * 16, 16)] = zeros

    def bias_f(f, carry):
        pltpu.async_copy(bias_tab.at[idxf_v.at[f]], bv_v, sem_c).wait()

        def addg(j, carry2):
            sl = pl.ds(j * 16, 16)
            acc_v[sl] = acc_v[sl] + bv_v[sl]
            return carry2

        lax.fori_loop(0, BPW // 16, addg, 0)
        return carry

    lax.fori_loop(0, F, bias_f, 0)
    pltpu.sync_copy(acc_v, first_out.at[pl.ds(wid * BPW, BPW)])


def _sc_gather(tabT, bias_tab, idx):
    mesh = plsc.VectorSubcoreMesh(core_axis_name="c", subcore_axis_name="s",
                                  num_cores=NC, num_subcores=NS)
    call = pl.kernel(
        _sc_gather_body,
        out_type=(
            jax.ShapeDtypeStruct((FE, B), jnp.float32),
            jax.ShapeDtypeStruct((B,), jnp.float32),
        ),
        mesh=mesh,
        scratch_types=[
            pltpu.VMEM((2, B), jnp.int32),
            pltpu.VMEM((F, BPW), jnp.int32),
            pltpu.VMEM((B,), jnp.int32),
            pltpu.VMEM((B,), jnp.int32),
            pltpu.VMEM((B,), jnp.float32),
            pltpu.VMEM((B,), jnp.float32),
            pltpu.VMEM((BPW,), jnp.float32),
            pltpu.VMEM((BPW,), jnp.float32),
            pltpu.SemaphoreType.DMA,
            pltpu.SemaphoreType.DMA,
            pltpu.SemaphoreType.DMA,
        ],
        compiler_params=pltpu.CompilerParams(use_tc_tiling_on_sc=False),
    )
    return call(tabT, bias_tab, idx)


def _mlp_body(xnum, catT, first, Wn, bn, W1, b1, W2, b2, W3, b3, out):
    nemb = jnp.dot(xnum[...], Wn[...], preferred_element_type=jnp.float32)
    nemb = nemb + bn[...]
    h = jnp.dot(nemb, W1[0:E, :], preferred_element_type=jnp.float32)
    h = h + lax.dot_general(catT[...], W1[E:D_IN, :],
                            dimension_numbers=(((0,), (0,)), ((), ())),
                            preferred_element_type=jnp.float32)
    h = jnp.maximum(h + b1[...], 0.0)
    h = jnp.maximum(jnp.dot(h, W2[...], preferred_element_type=jnp.float32) + b2[...], 0.0)
    deep = jnp.dot(h, W3[...], preferred_element_type=jnp.float32) + b3[...]
    out[:, 0:OUT] = deep
    out[:, OUT:OUT + 1] = first[...]


def _mlp(xnum, catT, first, Wn, bn, W1, b1, W2, b2, W3, b3):
    grid = (B // BT,)
    full = lambda shape: pl.BlockSpec(shape, lambda i: (0, 0))
    return pl.pallas_call(
        _mlp_body,
        grid=grid,
        in_specs=[
            pl.BlockSpec((BT, NUMD), lambda i: (i, 0)),
            pl.BlockSpec((FE, BT), lambda i: (0, i)),
            pl.BlockSpec((BT, 1), lambda i: (i, 0)),
            full((NUMD, E)),
            full((1, E)),
            full((D_IN, H1)),
            full((1, H1)),
            full((H1, H2)),
            full((1, H2)),
            full((H2, OUT)),
            full((1, OUT)),
        ],
        out_specs=pl.BlockSpec((BT, OUT + 1), lambda i: (i, 0)),
        out_shape=jax.ShapeDtypeStruct((B, OUT + 1), jnp.float32),
    )(xnum, catT, first, Wn, bn, W1, b1, W2, b2, W3, b3)


def kernel(num_features, cat_features, feature_bias, cat_tables,
           W_num, b_num, W1, b1, W2, b2, W3, b3):
    idx = cat_features.astype(jnp.int32)
    tabT = cat_tables.transpose(0, 2, 1).reshape(FE * V)
    bias_flat = feature_bias.reshape(F * V)

    embT, first = _sc_gather(tabT, bias_flat, idx)
    first = first.reshape(B, 1)

    return _mlp(num_features, embT, first,
                W_num, b_num.reshape(1, E), W1, b1.reshape(1, H1),
                W2, b2.reshape(1, H2), W3, b3.reshape(1, OUT))
